# baseline (speedup 1.0000x reference)
import functools

import jax
import jax.numpy as jnp
from jax.experimental import pallas as pl
from jax.experimental.pallas import tpu as pltpu

B = 4
N = 20000
C = 512
M_POINT = 1024
NUM_VIEW = 300
GRASPNESS_THRESHOLD = 0.1

NBLK = 5120


def _stage_a_body(w_ref, b_ref, f_ref, noise_ref, sel_ref):
    scores = jax.lax.dot_general(
        w_ref[...], f_ref[...], (((1,), (0,)), ((), ())),
        preferred_element_type=jnp.float32)
    s0 = scores[0, :] + b_ref[0, 0]
    s1 = scores[1, :] + b_ref[0, 1]
    s2 = scores[2, :] + b_ref[0, 2]
    mask = (s1 > s0) & (s2 > GRASPNESS_THRESHOLD)
    sel_ref[0, :] = noise_ref[0, :] + jnp.where(mask, 0.0, -2.0)


def _stage_a(seed_features, noise, W_graspable, b_graspable):
    w8 = jnp.zeros((8, C), jnp.float32).at[:3].set(W_graspable)
    b8 = jnp.zeros((1, 8), jnp.float32).at[0, :3].set(b_graspable)
    grid = (B, (N + NBLK - 1) // NBLK)
    return pl.pallas_call(
        _stage_a_body,
        grid=grid,
        in_specs=[
            pl.BlockSpec((8, C), lambda b, n: (0, 0)),
            pl.BlockSpec((1, 8), lambda b, n: (0, 0)),
            pl.BlockSpec((None, C, NBLK), lambda b, n: (b, 0, n)),
            pl.BlockSpec((None, 1, NBLK), lambda b, n: (b, 0, n)),
        ],
        out_specs=pl.BlockSpec((None, 1, NBLK), lambda b, n: (b, 0, n)),
        out_shape=jax.ShapeDtypeStruct((B, 1, N), jnp.float32),
    )(w8, b8, seed_features, noise.reshape(B, 1, N)).reshape(B, N)


V_PAD = 304


def _view_body(wv_ref, bv_ref, f_ref, vs_ref, am_ref):
    vs = jax.lax.dot_general(
        wv_ref[...], f_ref[...], (((1,), (0,)), ((), ())),
        preferred_element_type=jnp.float32) + bv_ref[0, :][:, None]
    vs_ref[...] = vs
    am_ref[0, :] = jnp.argmax(vs[:NUM_VIEW, :], axis=0).astype(jnp.int32)


def _view_head(feats_g, W_view, b_view):
    wv = jnp.zeros((V_PAD, C), jnp.float32).at[:NUM_VIEW].set(W_view)
    bv = jnp.zeros((1, V_PAD), jnp.float32).at[0, :NUM_VIEW].set(b_view)
    vs, am = pl.pallas_call(
        _view_body,
        grid=(B,),
        in_specs=[
            pl.BlockSpec((V_PAD, C), lambda b: (0, 0)),
            pl.BlockSpec((1, V_PAD), lambda b: (0, 0)),
            pl.BlockSpec((None, C, M_POINT), lambda b: (b, 0, 0)),
        ],
        out_specs=[
            pl.BlockSpec((None, V_PAD, M_POINT), lambda b: (b, 0, 0)),
            pl.BlockSpec((None, 1, M_POINT), lambda b: (b, 0, 0)),
        ],
        out_shape=[
            jax.ShapeDtypeStruct((B, V_PAD, M_POINT), jnp.float32),
            jax.ShapeDtypeStruct((B, 1, M_POINT), jnp.int32),
        ],
    )(wv, bv, feats_g)
    return vs[:, :NUM_VIEW, :], am.reshape(B, M_POINT)


def kernel(seed_xyz, seed_features, noise, W_graspable, b_graspable, W_view, b_view):
    sel = _stage_a(seed_features, noise, W_graspable, b_graspable)
    _, idxs = jax.lax.top_k(sel, M_POINT)
    seed_xyz_graspable = jnp.take_along_axis(seed_xyz, idxs[:, :, None], axis=1)
    feats_g = jnp.take_along_axis(seed_features, idxs[:, None, :], axis=2)
    view_score, grasp_top_view_inds = _view_head(feats_g, W_view, b_view)
    return view_score, seed_xyz_graspable, grasp_top_view_inds


# speedup vs baseline: 1.0007x; 1.0007x over previous
import jax
import jax.numpy as jnp
from jax.experimental import pallas as pl
from jax.experimental.pallas import tpu as pltpu

B = 4
N = 20000
C = 512
M_POINT = 1024
NUM_VIEW = 300
GRASPNESS_THRESHOLD = 0.1

NBLK = 5120


def _stage_a_body(w_ref, b_ref, f_ref, noise_ref, sel_ref):
    scores = jax.lax.dot_general(
        w_ref[...], f_ref[...], (((1,), (0,)), ((), ())),
        preferred_element_type=jnp.float32)
    s0 = scores[0, :] + b_ref[0, 0]
    s1 = scores[1, :] + b_ref[0, 1]
    s2 = scores[2, :] + b_ref[0, 2]
    mask = (s1 > s0) & (s2 > GRASPNESS_THRESHOLD)
    sel_ref[0, :] = noise_ref[0, :] + jnp.where(mask, 0.0, -2.0)


def _stage_a(seed_features, noise, W_graspable, b_graspable):
    w8 = jnp.zeros((8, C), jnp.float32).at[:3].set(W_graspable)
    b8 = jnp.zeros((1, 8), jnp.float32).at[0, :3].set(b_graspable)
    grid = (B, (N + NBLK - 1) // NBLK)
    return pl.pallas_call(
        _stage_a_body,
        grid=grid,
        in_specs=[
            pl.BlockSpec((8, C), lambda b, n: (0, 0)),
            pl.BlockSpec((1, 8), lambda b, n: (0, 0)),
            pl.BlockSpec((None, C, NBLK), lambda b, n: (b, 0, n)),
            pl.BlockSpec((None, 1, NBLK), lambda b, n: (b, 0, n)),
        ],
        out_specs=pl.BlockSpec((None, 1, NBLK), lambda b, n: (b, 0, n)),
        out_shape=jax.ShapeDtypeStruct((B, 1, N), jnp.float32),
    )(w8, b8, seed_features, noise.reshape(B, 1, N)).reshape(B, N)


V_PAD = 304


def _view_body(wv_ref, bv_ref, f_ref, vs_ref, am_ref):
    vs = jax.lax.dot_general(
        wv_ref[...], f_ref[...], (((1,), (0,)), ((), ())),
        preferred_element_type=jnp.float32) + bv_ref[0, :][:, None]
    vs_ref[...] = vs
    am_ref[0, :] = jnp.argmax(vs[:NUM_VIEW, :], axis=0).astype(jnp.int32)


def _view_head(feats_g, W_view, b_view):
    wv = jnp.zeros((V_PAD, C), jnp.float32).at[:NUM_VIEW].set(W_view)
    bv = jnp.zeros((1, V_PAD), jnp.float32).at[0, :NUM_VIEW].set(b_view)
    vs, am = pl.pallas_call(
        _view_body,
        grid=(B,),
        in_specs=[
            pl.BlockSpec((V_PAD, C), lambda b: (0, 0)),
            pl.BlockSpec((1, V_PAD), lambda b: (0, 0)),
            pl.BlockSpec((None, C, M_POINT), lambda b: (b, 0, 0)),
        ],
        out_specs=[
            pl.BlockSpec((None, V_PAD, M_POINT), lambda b: (b, 0, 0)),
            pl.BlockSpec((None, 1, M_POINT), lambda b: (b, 0, 0)),
        ],
        out_shape=[
            jax.ShapeDtypeStruct((B, V_PAD, M_POINT), jnp.float32),
            jax.ShapeDtypeStruct((B, 1, M_POINT), jnp.int32),
        ],
    )(wv, bv, feats_g)
    return vs[:, :NUM_VIEW, :], am.reshape(B, M_POINT)


def kernel(seed_xyz, seed_features, noise, W_graspable, b_graspable, W_view, b_view):
    sel = _stage_a(seed_features, noise, W_graspable, b_graspable)
    _, idxs = jax.lax.top_k(sel, M_POINT)
    seed_xyz_graspable = jnp.take_along_axis(seed_xyz, idxs[:, :, None], axis=1)
    feats_g = jnp.take_along_axis(seed_features, idxs[:, None, :], axis=2)
    view_score, grasp_top_view_inds = _view_head(feats_g, W_view, b_view)
    return view_score, seed_xyz_graspable, grasp_top_view_inds
